# Initial kernel scaffold; baseline (speedup 1.0000x reference)
import jax
import jax.numpy as jnp
from jax.experimental import pallas as pl


def kernel(x):
    raise NotImplementedError("write your pallas kernel here")


# speedup vs baseline: 2.6239x; 2.6239x over previous
import jax
import jax.numpy as jnp
from jax import lax
from jax.experimental import pallas as pl

_N = 128
_BB = 32


def _diag_kernel(x_ref, o_ref):
    i = lax.broadcasted_iota(jnp.int32, (_N, _N), 0)
    j = lax.broadcasted_iota(jnp.int32, (_N, _N), 1)
    mask = (i == j)[None]
    o_ref[...] = jnp.where(mask, x_ref[...][:, :, None], jnp.float32(0))


def kernel(x):
    b = x.shape[0]
    grid = (b // _BB,)
    return pl.pallas_call(
        _diag_kernel,
        grid=grid,
        in_specs=[pl.BlockSpec((_BB, _N), lambda g: (g, 0))],
        out_specs=pl.BlockSpec((_BB, _N, _N), lambda g: (g, 0, 0)),
        out_shape=jax.ShapeDtypeStruct((b, _N, _N), x.dtype),
    )(x)


# speedup vs baseline: 3.7578x; 1.4322x over previous
import jax
import jax.numpy as jnp
from jax import lax
from jax.experimental import pallas as pl

_N = 128
_BB = 128


def _diag_kernel(x_ref, o_ref):
    i = lax.broadcasted_iota(jnp.int32, (_N, _N), 0)
    j = lax.broadcasted_iota(jnp.int32, (_N, _N), 1)
    mask = (i == j)[None]
    o_ref[...] = jnp.where(mask, x_ref[...][:, :, None], jnp.float32(0))


def kernel(x):
    b = x.shape[0]
    grid = (b // _BB,)
    return pl.pallas_call(
        _diag_kernel,
        grid=grid,
        in_specs=[pl.BlockSpec((_BB, _N), lambda g: (g, 0))],
        out_specs=pl.BlockSpec((_BB, _N, _N), lambda g: (g, 0, 0)),
        out_shape=jax.ShapeDtypeStruct((b, _N, _N), x.dtype),
    )(x)
